# baseline (speedup 1.0000x reference)
import functools
import math

import jax
import jax.numpy as jnp
from jax import lax
from jax.experimental import pallas as pl
from jax.experimental.pallas import tpu as pltpu
from jax.experimental.pallas import tpu_sc as plsc

N = 100000
E = 1600000
U = 32
DEPTH = 12

PE = 4
LE = PE * U
PN = 4
LN = PN * U
NP_ = N // PN
EP_ = E // PE

NBLK = 1000
EBLK = 2000

SC_CORES = 2
SC_SUBCORES = 16
SC_WORKERS = SC_CORES * SC_SUBCORES
GCHUNK = 1024
GSUB = 128
NSUB = GCHUNK // GSUB
CHUNKS_PER_TILE = 50
TILE_E = CHUNKS_PER_TILE * GCHUNK
E_PAD = SC_WORKERS * TILE_E
EP_PAD = E_PAD // PE
SCAT_TILE_E = E_PAD // SC_SUBCORES
SCAT_CHUNKS = SCAT_TILE_E // GCHUNK
JUNK = N
N_SC = 100352
NP_SC = N_SC // PN
NROW_SC = N_SC // SC_SUBCORES

_seq = pltpu.CompilerParams(dimension_semantics=("arbitrary",))


def _diagp(w, p):
    return jnp.kron(jnp.eye(p, dtype=w.dtype), w)


def _tilep(b, p):
    return jnp.tile(b, p).reshape(1, p * U)


def _silu(z):
    return z * jax.nn.sigmoid(z)


def _init_body(lanes, xr, wr, br, outr):
    xb = xr[...]
    xe = jnp.broadcast_to(xb[:, :, None], xb.shape + (U,)).reshape(
        xb.shape[0], lanes)
    outr[...] = _silu(xe * wr[...] + br[...])


def _feat_init(x2d, w_row, bias, nrows, blk, p):
    grid = nrows // blk
    lanes = p * U
    return pl.pallas_call(
        functools.partial(_init_body, lanes),
        grid=(grid,),
        in_specs=[
            pl.BlockSpec((blk, p), lambda i: (i, 0)),
            pl.BlockSpec((1, lanes), lambda i: (0, 0)),
            pl.BlockSpec((1, lanes), lambda i: (0, 0)),
        ],
        out_specs=pl.BlockSpec((blk, lanes), lambda i: (i, 0)),
        out_shape=jax.ShapeDtypeStruct((nrows, lanes), jnp.float32),
        compiler_params=_seq,
    )(x2d, _tilep(w_row, p), _tilep(bias, p))


def _node_pre_body(hr, wb, wc, wv, wu, bb, bc, bv, bu, bxr, cxr, nvr, uhr):
    h = hr[...]
    bxr[...] = jnp.dot(h, wb[...], preferred_element_type=jnp.float32) + bb[...]
    cxr[...] = jnp.dot(h, wc[...], preferred_element_type=jnp.float32) + bc[...]
    nvr[...] = jnp.dot(h, wv[...], preferred_element_type=jnp.float32) + bv[...]
    uhr[...] = jnp.dot(h, wu[...], preferred_element_type=jnp.float32) + bu[...]


def _node_pre(h, wb4, wc4, wv4, wu4, bb4, bc4, bv4, bu4):
    grid = NP_ // NBLK
    blk = pl.BlockSpec((NBLK, LN), lambda i: (i, 0))
    wspec = pl.BlockSpec((LN, LN), lambda i: (0, 0))
    bspec = pl.BlockSpec((1, LN), lambda i: (0, 0))
    out = jax.ShapeDtypeStruct((NP_, LN), jnp.float32)
    return pl.pallas_call(
        _node_pre_body,
        grid=(grid,),
        in_specs=[blk, wspec, wspec, wspec, wspec, bspec, bspec, bspec, bspec],
        out_specs=[blk, blk, blk, blk],
        out_shape=[out, out, out, out],
        compiler_params=_seq,
    )(h, wb4, wc4, wv4, wu4, bb4, bc4, bv4, bu4)


def _e1_body(wr, gbr, gcr, wa, ba, einr, sumr):
    i = pl.program_id(0)
    ein = (jnp.dot(wr[...], wa[...], preferred_element_type=jnp.float32)
           + ba[...] + gbr[...] + gcr[...])
    einr[...] = ein

    @pl.when(i == 0)
    def _():
        sumr[...] = jnp.zeros_like(sumr)

    s1 = jnp.sum(ein, axis=0, keepdims=True)
    s2 = jnp.sum(ein * ein, axis=0, keepdims=True)
    sumr[...] += jnp.concatenate([s1, s2], axis=0)


def _edge1(w, gb, gc, wa8, ba8):
    grid = EP_ // EBLK
    blk = pl.BlockSpec((EBLK, LE), lambda i: (i, 0))
    return pl.pallas_call(
        _e1_body,
        grid=(grid,),
        in_specs=[blk, blk, blk,
                  pl.BlockSpec((LE, LE), lambda i: (0, 0)),
                  pl.BlockSpec((1, LE), lambda i: (0, 0))],
        out_specs=[blk, pl.BlockSpec((2, LE), lambda i: (0, 0))],
        out_shape=[jax.ShapeDtypeStruct((EP_, LE), jnp.float32),
                   jax.ShapeDtypeStruct((2, LE), jnp.float32)],
        compiler_params=_seq,
    )(w, gb, gc, wa8, ba8)


def _e2_body(wr, einr, gvr, scr, shr, wnr, msgr):
    tmp = jnp.maximum(einr[...] * scr[...] + shr[...], 0.0)
    wn = wr[...] + tmp
    wnr[...] = wn
    msgr[...] = jax.nn.sigmoid(wn) * gvr[...]


def _edge2(w, ein, gv, sc8, sh8):
    grid = EP_ // EBLK
    blk = pl.BlockSpec((EBLK, LE), lambda i: (i, 0))
    sspec = pl.BlockSpec((1, LE), lambda i: (0, 0))
    return pl.pallas_call(
        _e2_body,
        grid=(grid,),
        in_specs=[blk, blk, blk, sspec, sspec],
        out_specs=[blk, blk],
        out_shape=[jax.ShapeDtypeStruct((EP_, LE), jnp.float32),
                   jax.ShapeDtypeStruct((EP_PAD, LE), jnp.float32)],
        compiler_params=_seq,
    )(w, ein, gv, sc8, sh8)


def _nb1_body(uhr, aggr, sumr):
    i = pl.program_id(0)
    t = uhr[...] + aggr[...]

    @pl.when(i == 0)
    def _():
        sumr[...] = jnp.zeros_like(sumr)

    s1 = jnp.sum(t, axis=0, keepdims=True)
    s2 = jnp.sum(t * t, axis=0, keepdims=True)
    sumr[...] += jnp.concatenate([s1, s2], axis=0)


def _node_b1(uh, agg):
    grid = NP_ // NBLK
    blk = pl.BlockSpec((NBLK, LN), lambda i: (i, 0))
    return pl.pallas_call(
        _nb1_body,
        grid=(grid,),
        in_specs=[blk, blk],
        out_specs=pl.BlockSpec((2, LN), lambda i: (0, 0)),
        out_shape=jax.ShapeDtypeStruct((2, LN), jnp.float32),
        compiler_params=_seq,
    )(uh, agg)


def _nb2_body(hr, uhr, aggr, scr, shr, outr):
    t = uhr[...] + aggr[...]
    outr[...] = hr[...] + jnp.maximum(t * scr[...] + shr[...], 0.0)


def _node_b2(h, uh, agg, sc4, sh4):
    grid = NP_ // NBLK
    blk = pl.BlockSpec((NBLK, LN), lambda i: (i, 0))
    sspec = pl.BlockSpec((1, LN), lambda i: (0, 0))
    return pl.pallas_call(
        _nb2_body,
        grid=(grid,),
        in_specs=[blk, blk, blk, sspec, sspec],
        out_specs=blk,
        out_shape=jax.ShapeDtypeStruct((NP_, LN), jnp.float32),
        compiler_params=_seq,
    )(h, uh, agg, sc4, sh4)


def _readout_body(wr, w1, b1, w2, b2, w3, b3r, outr):
    t = _silu(jnp.dot(wr[...], w1[...], preferred_element_type=jnp.float32)
              + b1[...])
    t = _silu(jnp.dot(t, w2[...], preferred_element_type=jnp.float32)
              + b2[...])
    z = jnp.dot(t, w3[...], preferred_element_type=jnp.float32) + b3r[0]
    outr[...] = jax.nn.sigmoid(z)


def _readout(w, w18, b18, w28, b28, w38, b3):
    grid = EP_ // EBLK
    blk = pl.BlockSpec((EBLK, LE), lambda i: (i, 0))
    return pl.pallas_call(
        _readout_body,
        grid=(grid,),
        in_specs=[blk,
                  pl.BlockSpec((LE, LE), lambda i: (0, 0)),
                  pl.BlockSpec((1, LE), lambda i: (0, 0)),
                  pl.BlockSpec((LE, LE), lambda i: (0, 0)),
                  pl.BlockSpec((1, LE), lambda i: (0, 0)),
                  pl.BlockSpec((LE, PE), lambda i: (0, 0)),
                  pl.BlockSpec(memory_space=pltpu.SMEM)],
        out_specs=pl.BlockSpec((EBLK, PE), lambda i: (i, 0)),
        out_shape=jax.ShapeDtypeStruct((EP_, PE), jnp.float32),
        compiler_params=_seq,
    )(w, w18, b18, w28, b28, w38, b3)


def _bn_coeffs(sums, count, g, be, p):
    s = sums.reshape(2, p, U).sum(axis=1)
    m = s[0] / count
    v = s[1] / count - m * m
    scale = g * lax.rsqrt(v + 1e-5)
    shift = be - m * scale
    return _tilep(scale, p), _tilep(shift, p)


def _sc_gather(bx, cx, nv, rowi, coli):
    out = jax.ShapeDtypeStruct((E_PAD, U), jnp.float32)

    @functools.partial(
        pl.kernel,
        out_type=[out, out, out],
        mesh=plsc.VectorSubcoreMesh(core_axis_name="c", subcore_axis_name="s"),
        scratch_types=[
            pltpu.VMEM((NSUB, GSUB), jnp.int32),
            pltpu.VMEM((NSUB, GSUB), jnp.int32),
            pltpu.VMEM((GCHUNK, U), jnp.float32),
            pltpu.VMEM((GCHUNK, U), jnp.float32),
            pltpu.VMEM((GCHUNK, U), jnp.float32),
            pltpu.SemaphoreType.DMA,
        ],
        compiler_params=pltpu.CompilerParams(use_tc_tiling_on_sc=False),
    )
    def body(bxr, cxr, nvr, rowr, colr, gbo, gco, gvo,
             rowv, colv, b1, b2, b3, sem):
        c = lax.axis_index("c")
        s = lax.axis_index("s")
        wid = c * SC_SUBCORES + s

        def chunk(ci, carry):
            ibase = wid * (TILE_E // GSUB) + ci * NSUB
            pltpu.sync_copy(rowr.at[pl.ds(ibase, NSUB)], rowv)
            pltpu.sync_copy(colr.at[pl.ds(ibase, NSUB)], colv)
            descs = []
            for j in range(NSUB):
                dst = pl.ds(j * GSUB, GSUB)
                descs.append(pltpu.async_copy(
                    bxr.at[rowv.at[j]], b1.at[dst], sem))
                descs.append(pltpu.async_copy(
                    cxr.at[colv.at[j]], b2.at[dst], sem))
                descs.append(pltpu.async_copy(
                    nvr.at[colv.at[j]], b3.at[dst], sem))
            for d in descs:
                d.wait()
            e0 = wid * TILE_E + ci * GCHUNK
            pltpu.sync_copy(b1, gbo.at[pl.ds(e0, GCHUNK)])
            pltpu.sync_copy(b2, gco.at[pl.ds(e0, GCHUNK)])
            pltpu.sync_copy(b3, gvo.at[pl.ds(e0, GCHUNK)])
            return carry

        lax.fori_loop(0, CHUNKS_PER_TILE, chunk, 0)

    return body(bx, cx, nv, rowi, coli)


def _sc_scatter(msg, rowi, zeros16):
    @functools.partial(
        pl.kernel,
        out_type=jax.ShapeDtypeStruct((N_SC, U), jnp.float32),
        mesh=plsc.VectorSubcoreMesh(core_axis_name="c", subcore_axis_name="s"),
        scratch_types=[
            pltpu.VMEM((NSUB, GSUB), jnp.int32),
            pltpu.VMEM((GCHUNK, U // 2), jnp.float32),
            pltpu.VMEM_SHARED((N_SC, U // 2), jnp.float32),
            pltpu.SemaphoreType.DMA,
        ],
        compiler_params=pltpu.CompilerParams(use_tc_tiling_on_sc=False),
    )
    def body(msgr, rowr, zr, aggo, idxv, mbuf, aggsh, ssem):
        c = lax.axis_index("c")
        s = lax.axis_index("s")
        half = U // 2
        r0 = s * NROW_SC
        pltpu.sync_copy(zr.at[pl.ds(r0, NROW_SC)],
                        aggsh.at[pl.ds(r0, NROW_SC)])
        plsc.subcore_barrier()

        def chunk(ci, carry):
            e0 = s * SCAT_TILE_E + ci * GCHUNK
            pltpu.sync_copy(rowr.at[pl.ds(e0 // GSUB, NSUB)], idxv)
            pltpu.sync_copy(
                msgr.at[pl.ds(e0, GCHUNK), pl.ds(c * half, half)], mbuf)
            for j in range(NSUB):
                pltpu.sync_copy(mbuf.at[pl.ds(j * GSUB, GSUB)],
                                aggsh.at[idxv.at[j]], add=True)
            return carry

        lax.fori_loop(0, SCAT_CHUNKS, chunk, 0)
        plsc.subcore_barrier()
        pltpu.sync_copy(
            aggsh.at[pl.ds(r0, NROW_SC)],
            aggo.at[pl.ds(r0, NROW_SC), pl.ds(c * half, half)])

    return body(msg, rowi, zeros16)


def kernel(x, edge_attr, edge_index, W0v, b0v, W0e, b0e, Wu, bu, Wv, bv,
           WA, bA, WB, bB, WC, bC, g_node, be_node, g_edge, be_edge,
           Wp1, bp1, Wp2, bp2, Wp3, bp3):
    row = edge_index[0]
    col = edge_index[1]

    gpad = jnp.zeros((E_PAD - E,), jnp.int32)
    rowg = jnp.concatenate([row, gpad]).reshape(E_PAD // GSUB, GSUB)
    colg = jnp.concatenate([col, gpad]).reshape(E_PAD // GSUB, GSUB)
    rows_sc = jnp.concatenate(
        [row, jnp.full((E_PAD - E,), JUNK, jnp.int32)]
    ).reshape(E_PAD // GSUB, GSUB)
    zeros16 = jnp.zeros((N_SC, U // 2), jnp.float32)

    x2 = x.reshape(NP_, PN)
    ea2 = edge_attr.reshape(EP_, PE)

    h = _feat_init(x2, W0v[0], b0v, NP_, NBLK, PN)
    w = _feat_init(ea2, W0e[0], b0e, EP_, EBLK, PE)

    wu4 = jax.vmap(lambda m: _diagp(m, PN))(Wu)
    wv4 = jax.vmap(lambda m: _diagp(m, PN))(Wv)
    wb4 = jax.vmap(lambda m: _diagp(m, PN))(WB)
    wc4 = jax.vmap(lambda m: _diagp(m, PN))(WC)
    wa8 = jax.vmap(lambda m: _diagp(m, PE))(WA)

    for i in range(DEPTH):
        bx, cx, nv, uh = _node_pre(
            h, wb4[i], wc4[i], wv4[i], wu4[i],
            _tilep(bB[i], PN), _tilep(bC[i], PN),
            _tilep(bv[i], PN), _tilep(bu[i], PN))

        gb, gc, gv = _sc_gather(bx.reshape(N, U), cx.reshape(N, U),
                                nv.reshape(N, U), rowg, colg)
        gb = gb.reshape(EP_PAD, LE)
        gc = gc.reshape(EP_PAD, LE)
        gv = gv.reshape(EP_PAD, LE)

        ein, esums = _edge1(w, gb, gc, wa8[i], _tilep(bA[i], PE))
        esc, esh = _bn_coeffs(esums, float(E), g_edge[i], be_edge[i], PE)
        w, msg = _edge2(w, ein, gv, esc, esh)

        agg = _sc_scatter(msg.reshape(E_PAD, U), rows_sc, zeros16)
        agg = agg.reshape(NP_SC, LN)

        nsums = _node_b1(uh, agg)
        nsc, nsh = _bn_coeffs(nsums, float(N), g_node[i], be_node[i], PN)
        h = _node_b2(h, uh, agg, nsc, nsh)

    p = _readout(w, _diagp(Wp1, PE), _tilep(bp1, PE),
                 _diagp(Wp2, PE), _tilep(bp2, PE), _diagp(Wp3, PE), bp3)
    return p.reshape(E)


# speedup vs baseline: 1.3529x; 1.3529x over previous
import functools
import math

import jax
import jax.numpy as jnp
from jax import lax
from jax.experimental import pallas as pl
from jax.experimental.pallas import tpu as pltpu
from jax.experimental.pallas import tpu_sc as plsc

N = 100000
E = 1600000
U = 32
DEPTH = 12

PE = 4
LE = PE * U
PN = 4
LN = PN * U
NP_ = N // PN
EP_ = E // PE

NBLK = 1000
EBLK = 2000

SC_CORES = 2
SC_SUBCORES = 16
SC_WORKERS = SC_CORES * SC_SUBCORES
GCHUNK = 1024
GSUB = 128
NSUB = GCHUNK // GSUB
CHUNKS_PER_TILE = 49
TILE_E = CHUNKS_PER_TILE * GCHUNK
E_PAD = SC_WORKERS * TILE_E
EP_PAD = E_PAD // PE
SCAT_TILE_E = E_PAD // SC_SUBCORES
SCAT_CHUNKS = SCAT_TILE_E // GCHUNK
JUNK = N
N_SC = 100352
NP_SC = N_SC // PN
NROW_SC = N_SC // SC_SUBCORES

_seq = pltpu.CompilerParams(dimension_semantics=("arbitrary",))


def _diagp(w, p):
    return jnp.kron(jnp.eye(p, dtype=w.dtype), w)


def _tilep(b, p):
    return jnp.tile(b, p).reshape(1, p * U)


def _silu(z):
    return z * jax.nn.sigmoid(z)


def _init_body(lanes, xr, wr, br, outr):
    xb = xr[...]
    xe = jnp.broadcast_to(xb[:, :, None], xb.shape + (U,)).reshape(
        xb.shape[0], lanes)
    outr[...] = _silu(xe * wr[...] + br[...])


def _feat_init(x2d, w_row, bias, nrows, blk, p):
    grid = nrows // blk
    lanes = p * U
    return pl.pallas_call(
        functools.partial(_init_body, lanes),
        grid=(grid,),
        in_specs=[
            pl.BlockSpec((blk, p), lambda i: (i, 0)),
            pl.BlockSpec((1, lanes), lambda i: (0, 0)),
            pl.BlockSpec((1, lanes), lambda i: (0, 0)),
        ],
        out_specs=pl.BlockSpec((blk, lanes), lambda i: (i, 0)),
        out_shape=jax.ShapeDtypeStruct((nrows, lanes), jnp.float32),
        compiler_params=_seq,
    )(x2d, _tilep(w_row, p), _tilep(bias, p))


def _node_pre_body(hr, wb, wc, wv, wu, bb, bc, bv, bu, bxr, cxr, nvr, uhr):
    h = hr[...]
    bxr[...] = jnp.dot(h, wb[...], preferred_element_type=jnp.float32) + bb[...]
    cxr[...] = jnp.dot(h, wc[...], preferred_element_type=jnp.float32) + bc[...]
    nvr[...] = jnp.dot(h, wv[...], preferred_element_type=jnp.float32) + bv[...]
    uhr[...] = jnp.dot(h, wu[...], preferred_element_type=jnp.float32) + bu[...]


def _node_pre(h, wb4, wc4, wv4, wu4, bb4, bc4, bv4, bu4):
    grid = NP_ // NBLK
    blk = pl.BlockSpec((NBLK, LN), lambda i: (i, 0))
    wspec = pl.BlockSpec((LN, LN), lambda i: (0, 0))
    bspec = pl.BlockSpec((1, LN), lambda i: (0, 0))
    out = jax.ShapeDtypeStruct((NP_, LN), jnp.float32)
    return pl.pallas_call(
        _node_pre_body,
        grid=(grid,),
        in_specs=[blk, wspec, wspec, wspec, wspec, bspec, bspec, bspec, bspec],
        out_specs=[blk, blk, blk, blk],
        out_shape=[out, out, out, out],
        compiler_params=_seq,
    )(h, wb4, wc4, wv4, wu4, bb4, bc4, bv4, bu4)


def _e1_body(wr, gbr, gcr, wa, ba, einr, sumr):
    i = pl.program_id(0)
    ein = (jnp.dot(wr[...], wa[...], preferred_element_type=jnp.float32)
           + ba[...] + gbr[...] + gcr[...])
    einr[...] = ein

    @pl.when(i == 0)
    def _():
        sumr[...] = jnp.zeros_like(sumr)

    s1 = jnp.sum(ein, axis=0, keepdims=True)
    s2 = jnp.sum(ein * ein, axis=0, keepdims=True)
    sumr[...] += jnp.concatenate([s1, s2], axis=0)


def _edge1(w, gb, gc, wa8, ba8):
    grid = EP_ // EBLK
    blk = pl.BlockSpec((EBLK, LE), lambda i: (i, 0))
    return pl.pallas_call(
        _e1_body,
        grid=(grid,),
        in_specs=[blk, blk, blk,
                  pl.BlockSpec((LE, LE), lambda i: (0, 0)),
                  pl.BlockSpec((1, LE), lambda i: (0, 0))],
        out_specs=[blk, pl.BlockSpec((2, LE), lambda i: (0, 0))],
        out_shape=[jax.ShapeDtypeStruct((EP_, LE), jnp.float32),
                   jax.ShapeDtypeStruct((2, LE), jnp.float32)],
        compiler_params=_seq,
    )(w, gb, gc, wa8, ba8)


def _e2_body(wr, einr, gvr, scr, shr, wnr, msgr):
    tmp = jnp.maximum(einr[...] * scr[...] + shr[...], 0.0)
    wn = wr[...] + tmp
    wnr[...] = wn
    msgr[...] = jax.nn.sigmoid(wn) * gvr[...]


def _edge2(w, ein, gv, sc8, sh8):
    grid = EP_ // EBLK
    blk = pl.BlockSpec((EBLK, LE), lambda i: (i, 0))
    sspec = pl.BlockSpec((1, LE), lambda i: (0, 0))
    return pl.pallas_call(
        _e2_body,
        grid=(grid,),
        in_specs=[blk, blk, blk, sspec, sspec],
        out_specs=[blk, blk],
        out_shape=[jax.ShapeDtypeStruct((EP_, LE), jnp.float32),
                   jax.ShapeDtypeStruct((EP_PAD, LE), jnp.float32)],
        compiler_params=_seq,
    )(w, ein, gv, sc8, sh8)


def _nb1_body(uhr, aggr, sumr):
    i = pl.program_id(0)
    t = uhr[...] + aggr[...]

    @pl.when(i == 0)
    def _():
        sumr[...] = jnp.zeros_like(sumr)

    s1 = jnp.sum(t, axis=0, keepdims=True)
    s2 = jnp.sum(t * t, axis=0, keepdims=True)
    sumr[...] += jnp.concatenate([s1, s2], axis=0)


def _node_b1(uh, agg):
    grid = NP_ // NBLK
    blk = pl.BlockSpec((NBLK, LN), lambda i: (i, 0))
    return pl.pallas_call(
        _nb1_body,
        grid=(grid,),
        in_specs=[blk, blk],
        out_specs=pl.BlockSpec((2, LN), lambda i: (0, 0)),
        out_shape=jax.ShapeDtypeStruct((2, LN), jnp.float32),
        compiler_params=_seq,
    )(uh, agg)


def _nb2_body(hr, uhr, aggr, scr, shr, outr):
    t = uhr[...] + aggr[...]
    outr[...] = hr[...] + jnp.maximum(t * scr[...] + shr[...], 0.0)


def _node_b2(h, uh, agg, sc4, sh4):
    grid = NP_ // NBLK
    blk = pl.BlockSpec((NBLK, LN), lambda i: (i, 0))
    sspec = pl.BlockSpec((1, LN), lambda i: (0, 0))
    return pl.pallas_call(
        _nb2_body,
        grid=(grid,),
        in_specs=[blk, blk, blk, sspec, sspec],
        out_specs=blk,
        out_shape=jax.ShapeDtypeStruct((NP_, LN), jnp.float32),
        compiler_params=_seq,
    )(h, uh, agg, sc4, sh4)


def _readout_body(wr, w1, b1, w2, b2, w3, b3r, outr):
    t = _silu(jnp.dot(wr[...], w1[...], preferred_element_type=jnp.float32)
              + b1[...])
    t = _silu(jnp.dot(t, w2[...], preferred_element_type=jnp.float32)
              + b2[...])
    z = jnp.dot(t, w3[...], preferred_element_type=jnp.float32) + b3r[0]
    outr[...] = jax.nn.sigmoid(z)


def _readout(w, w18, b18, w28, b28, w38, b3):
    grid = EP_ // EBLK
    blk = pl.BlockSpec((EBLK, LE), lambda i: (i, 0))
    return pl.pallas_call(
        _readout_body,
        grid=(grid,),
        in_specs=[blk,
                  pl.BlockSpec((LE, LE), lambda i: (0, 0)),
                  pl.BlockSpec((1, LE), lambda i: (0, 0)),
                  pl.BlockSpec((LE, LE), lambda i: (0, 0)),
                  pl.BlockSpec((1, LE), lambda i: (0, 0)),
                  pl.BlockSpec((LE, PE), lambda i: (0, 0)),
                  pl.BlockSpec(memory_space=pltpu.SMEM)],
        out_specs=pl.BlockSpec((EBLK, PE), lambda i: (i, 0)),
        out_shape=jax.ShapeDtypeStruct((EP_, PE), jnp.float32),
        compiler_params=_seq,
    )(w, w18, b18, w28, b28, w38, b3)


def _bn_coeffs(sums, count, g, be, p):
    s = sums.reshape(2, p, U).sum(axis=1)
    m = s[0] / count
    v = s[1] / count - m * m
    scale = g * lax.rsqrt(v + 1e-5)
    shift = be - m * scale
    return _tilep(scale, p), _tilep(shift, p)


def _sc_gather(bx, cx, nv, rowi, coli):
    out = jax.ShapeDtypeStruct((E_PAD, U), jnp.float32)

    @functools.partial(
        pl.kernel,
        out_type=[out, out, out],
        mesh=plsc.VectorSubcoreMesh(core_axis_name="c", subcore_axis_name="s"),
        scratch_types=[
            pltpu.VMEM((NSUB, GSUB), jnp.int32),
            pltpu.VMEM((NSUB, GSUB), jnp.int32),
            pltpu.VMEM((GCHUNK, U), jnp.float32),
            pltpu.VMEM((GCHUNK, U), jnp.float32),
            pltpu.VMEM((GCHUNK, U), jnp.float32),
            pltpu.SemaphoreType.DMA,
        ],
        compiler_params=pltpu.CompilerParams(use_tc_tiling_on_sc=False),
    )
    def body(bxr, cxr, nvr, rowr, colr, gbo, gco, gvo,
             rowv, colv, b1, b2, b3, sem):
        c = lax.axis_index("c")
        s = lax.axis_index("s")
        wid = c * SC_SUBCORES + s

        def chunk(ci, carry):
            ibase = wid * (TILE_E // GSUB) + ci * NSUB
            pltpu.sync_copy(rowr.at[pl.ds(ibase, NSUB)], rowv)
            pltpu.sync_copy(colr.at[pl.ds(ibase, NSUB)], colv)
            descs = []
            for j in range(NSUB):
                dst = pl.ds(j * GSUB, GSUB)
                descs.append(pltpu.async_copy(
                    bxr.at[rowv.at[j]], b1.at[dst], sem))
                descs.append(pltpu.async_copy(
                    cxr.at[colv.at[j]], b2.at[dst], sem))
                descs.append(pltpu.async_copy(
                    nvr.at[colv.at[j]], b3.at[dst], sem))
            for d in descs:
                d.wait()
            e0 = wid * TILE_E + ci * GCHUNK
            pltpu.sync_copy(b1, gbo.at[pl.ds(e0, GCHUNK)])
            pltpu.sync_copy(b2, gco.at[pl.ds(e0, GCHUNK)])
            pltpu.sync_copy(b3, gvo.at[pl.ds(e0, GCHUNK)])
            return carry

        lax.fori_loop(0, CHUNKS_PER_TILE, chunk, 0)

    return body(bx, cx, nv, rowi, coli)


def _sc_scatter(msg, rowi, zeros16):
    @functools.partial(
        pl.kernel,
        out_type=jax.ShapeDtypeStruct((N_SC, U), jnp.float32),
        mesh=plsc.VectorSubcoreMesh(core_axis_name="c", subcore_axis_name="s"),
        scratch_types=[
            pltpu.VMEM((NSUB, GSUB), jnp.int32),
            pltpu.VMEM((GCHUNK, U // 2), jnp.float32),
            pltpu.VMEM_SHARED((N_SC, U // 2), jnp.float32),
            pltpu.SemaphoreType.DMA,
        ],
        compiler_params=pltpu.CompilerParams(use_tc_tiling_on_sc=False),
    )
    def body(msgr, rowr, zr, aggo, idxv, mbuf, aggsh, ssem):
        c = lax.axis_index("c")
        s = lax.axis_index("s")
        half = U // 2
        r0 = s * NROW_SC
        pltpu.sync_copy(zr.at[pl.ds(r0, NROW_SC)],
                        aggsh.at[pl.ds(r0, NROW_SC)])
        plsc.subcore_barrier()

        def chunk(ci, carry):
            e0 = s * SCAT_TILE_E + ci * GCHUNK
            pltpu.sync_copy(rowr.at[pl.ds(e0 // GSUB, NSUB)], idxv)
            pltpu.sync_copy(
                msgr.at[pl.ds(e0, GCHUNK), pl.ds(c * half, half)], mbuf)
            for j in range(NSUB):
                pltpu.sync_copy(mbuf.at[pl.ds(j * GSUB, GSUB)],
                                aggsh.at[idxv.at[j]], add=True)
            return carry

        lax.fori_loop(0, SCAT_CHUNKS, chunk, 0)
        plsc.subcore_barrier()
        pltpu.sync_copy(
            aggsh.at[pl.ds(r0, NROW_SC)],
            aggo.at[pl.ds(r0, NROW_SC), pl.ds(c * half, half)])

    return body(msg, rowi, zeros16)


def kernel(x, edge_attr, edge_index, W0v, b0v, W0e, b0e, Wu, bu, Wv, bv,
           WA, bA, WB, bB, WC, bC, g_node, be_node, g_edge, be_edge,
           Wp1, bp1, Wp2, bp2, Wp3, bp3):
    row = edge_index[0]
    col = edge_index[1]

    gpad = jnp.zeros((E_PAD - E,), jnp.int32)
    rowg = jnp.concatenate([row, gpad]).reshape(E_PAD // GSUB, GSUB)
    colg = jnp.concatenate([col, gpad]).reshape(E_PAD // GSUB, GSUB)
    rows_sc = jnp.concatenate(
        [row, jnp.full((E_PAD - E,), JUNK, jnp.int32)]
    ).reshape(E_PAD // GSUB, GSUB)
    zeros16 = jnp.zeros((N_SC, U // 2), jnp.float32)

    x2 = x.reshape(NP_, PN)
    ea2 = edge_attr.reshape(EP_, PE)

    h = _feat_init(x2, W0v[0], b0v, NP_, NBLK, PN)
    w = _feat_init(ea2, W0e[0], b0e, EP_, EBLK, PE)

    wu4 = jax.vmap(lambda m: _diagp(m, PN))(Wu)
    wv4 = jax.vmap(lambda m: _diagp(m, PN))(Wv)
    wb4 = jax.vmap(lambda m: _diagp(m, PN))(WB)
    wc4 = jax.vmap(lambda m: _diagp(m, PN))(WC)
    wa8 = jax.vmap(lambda m: _diagp(m, PE))(WA)

    for i in range(DEPTH):
        bx, cx, nv, uh = _node_pre(
            h, wb4[i], wc4[i], wv4[i], wu4[i],
            _tilep(bB[i], PN), _tilep(bC[i], PN),
            _tilep(bv[i], PN), _tilep(bu[i], PN))

        gb, gc, gv = _sc_gather(bx.reshape(N, U), cx.reshape(N, U),
                                nv.reshape(N, U), rowg, colg)
        gb = gb.reshape(EP_PAD, LE)
        gc = gc.reshape(EP_PAD, LE)
        gv = gv.reshape(EP_PAD, LE)

        ein, esums = _edge1(w, gb, gc, wa8[i], _tilep(bA[i], PE))
        esc, esh = _bn_coeffs(esums, float(E), g_edge[i], be_edge[i], PE)
        w, msg = _edge2(w, ein, gv, esc, esh)

        agg = _sc_scatter(msg.reshape(E_PAD, U), rows_sc, zeros16)
        agg = agg.reshape(NP_SC, LN)

        nsums = _node_b1(uh, agg)
        nsc, nsh = _bn_coeffs(nsums, float(N), g_node[i], be_node[i], PN)
        h = _node_b2(h, uh, agg, nsc, nsh)

    p = _readout(w, _diagp(Wp1, PE), _tilep(bp1, PE),
                 _diagp(Wp2, PE), _tilep(bp2, PE), _diagp(Wp3, PE), bp3)
    return p.reshape(E)


# speedup vs baseline: 1.3855x; 1.0240x over previous
import functools
import math

import jax
import jax.numpy as jnp
from jax import lax
from jax.experimental import pallas as pl
from jax.experimental.pallas import tpu as pltpu
from jax.experimental.pallas import tpu_sc as plsc

N = 100000
E = 1600000
U = 32
DEPTH = 12

PE = 4
LE = PE * U
PN = 4
LN = PN * U
NP_ = N // PN
EP_ = E // PE

NBLK = 1000
EBLK = 2000

SC_CORES = 2
SC_SUBCORES = 16
SC_WORKERS = SC_CORES * SC_SUBCORES
GCHUNK = 1024
GSUB = 128
NSUB = GCHUNK // GSUB
CHUNKS_PER_TILE = 49
TILE_E = CHUNKS_PER_TILE * GCHUNK
E_PAD = SC_WORKERS * TILE_E
EP_PAD = E_PAD // PE
SCAT_TILE_E = E_PAD // SC_SUBCORES
SCAT_CHUNKS = SCAT_TILE_E // GCHUNK
JUNK = N
N_SC = 100352
NP_SC = N_SC // PN
NROW_SC = N_SC // SC_SUBCORES

_seq = pltpu.CompilerParams(dimension_semantics=("arbitrary",))


def _diagp(w, p):
    return jnp.kron(jnp.eye(p, dtype=w.dtype), w)


def _tilep(b, p):
    return jnp.tile(b, p).reshape(1, p * U)


def _silu(z):
    return z * jax.nn.sigmoid(z)


def _init_body(lanes, xr, wr, br, outr):
    xb = xr[...]
    xe = jnp.broadcast_to(xb[:, :, None], xb.shape + (U,)).reshape(
        xb.shape[0], lanes)
    outr[...] = _silu(xe * wr[...] + br[...])


def _feat_init(x2d, w_row, bias, nrows, blk, p):
    grid = nrows // blk
    lanes = p * U
    return pl.pallas_call(
        functools.partial(_init_body, lanes),
        grid=(grid,),
        in_specs=[
            pl.BlockSpec((blk, p), lambda i: (i, 0)),
            pl.BlockSpec((1, lanes), lambda i: (0, 0)),
            pl.BlockSpec((1, lanes), lambda i: (0, 0)),
        ],
        out_specs=pl.BlockSpec((blk, lanes), lambda i: (i, 0)),
        out_shape=jax.ShapeDtypeStruct((nrows, lanes), jnp.float32),
        compiler_params=_seq,
    )(x2d, _tilep(w_row, p), _tilep(bias, p))


def _node_pre_body(hr, wb, wc, wv, wu, bb, bc, bv, bu, bxr, cxr, nvr, uhr):
    h = hr[...]
    bxr[...] = jnp.dot(h, wb[...], preferred_element_type=jnp.float32) + bb[...]
    cxr[...] = jnp.dot(h, wc[...], preferred_element_type=jnp.float32) + bc[...]
    nvr[...] = jnp.dot(h, wv[...], preferred_element_type=jnp.float32) + bv[...]
    uhr[...] = jnp.dot(h, wu[...], preferred_element_type=jnp.float32) + bu[...]


def _node_pre(h, wb4, wc4, wv4, wu4, bb4, bc4, bv4, bu4):
    grid = NP_ // NBLK
    blk = pl.BlockSpec((NBLK, LN), lambda i: (i, 0))
    wspec = pl.BlockSpec((LN, LN), lambda i: (0, 0))
    bspec = pl.BlockSpec((1, LN), lambda i: (0, 0))
    out = jax.ShapeDtypeStruct((NP_, LN), jnp.float32)
    return pl.pallas_call(
        _node_pre_body,
        grid=(grid,),
        in_specs=[blk, wspec, wspec, wspec, wspec, bspec, bspec, bspec, bspec],
        out_specs=[blk, blk, blk, blk],
        out_shape=[out, out, out, out],
        compiler_params=_seq,
    )(h, wb4, wc4, wv4, wu4, bb4, bc4, bv4, bu4)


def _e1_body(wr, gbr, gcr, wa, ba, einr, sumr):
    i = pl.program_id(0)
    ein = (jnp.dot(wr[...], wa[...], preferred_element_type=jnp.float32)
           + ba[...] + gbr[...] + gcr[...])
    einr[...] = ein

    @pl.when(i == 0)
    def _():
        sumr[...] = jnp.zeros_like(sumr)

    s1 = jnp.sum(ein, axis=0, keepdims=True)
    s2 = jnp.sum(ein * ein, axis=0, keepdims=True)
    sumr[...] += jnp.concatenate([s1, s2], axis=0)


def _edge1(w, gb, gc, wa8, ba8):
    grid = EP_ // EBLK
    blk = pl.BlockSpec((EBLK, LE), lambda i: (i, 0))
    return pl.pallas_call(
        _e1_body,
        grid=(grid,),
        in_specs=[blk, blk, blk,
                  pl.BlockSpec((LE, LE), lambda i: (0, 0)),
                  pl.BlockSpec((1, LE), lambda i: (0, 0))],
        out_specs=[blk, pl.BlockSpec((2, LE), lambda i: (0, 0))],
        out_shape=[jax.ShapeDtypeStruct((EP_, LE), jnp.float32),
                   jax.ShapeDtypeStruct((2, LE), jnp.float32)],
        compiler_params=_seq,
    )(w, gb, gc, wa8, ba8)


def _e2_body(wr, einr, gvr, scr, shr, wnr, msgr):
    tmp = jnp.maximum(einr[...] * scr[...] + shr[...], 0.0)
    wn = wr[...] + tmp
    wnr[...] = wn
    msgr[...] = jax.nn.sigmoid(wn) * gvr[...]


def _edge2(w, ein, gv, sc8, sh8):
    grid = EP_ // EBLK
    blk = pl.BlockSpec((EBLK, LE), lambda i: (i, 0))
    sspec = pl.BlockSpec((1, LE), lambda i: (0, 0))
    return pl.pallas_call(
        _e2_body,
        grid=(grid,),
        in_specs=[blk, blk, blk, sspec, sspec],
        out_specs=[blk, blk],
        out_shape=[jax.ShapeDtypeStruct((EP_, LE), jnp.float32),
                   jax.ShapeDtypeStruct((EP_PAD, LE), jnp.float32)],
        compiler_params=_seq,
    )(w, ein, gv, sc8, sh8)


def _nb1_body(uhr, aggr, sumr):
    i = pl.program_id(0)
    t = uhr[...] + aggr[...]

    @pl.when(i == 0)
    def _():
        sumr[...] = jnp.zeros_like(sumr)

    s1 = jnp.sum(t, axis=0, keepdims=True)
    s2 = jnp.sum(t * t, axis=0, keepdims=True)
    sumr[...] += jnp.concatenate([s1, s2], axis=0)


def _node_b1(uh, agg):
    grid = NP_ // NBLK
    blk = pl.BlockSpec((NBLK, LN), lambda i: (i, 0))
    return pl.pallas_call(
        _nb1_body,
        grid=(grid,),
        in_specs=[blk, blk],
        out_specs=pl.BlockSpec((2, LN), lambda i: (0, 0)),
        out_shape=jax.ShapeDtypeStruct((2, LN), jnp.float32),
        compiler_params=_seq,
    )(uh, agg)


def _nb2_body(hr, uhr, aggr, scr, shr, outr):
    t = uhr[...] + aggr[...]
    outr[...] = hr[...] + jnp.maximum(t * scr[...] + shr[...], 0.0)


def _node_b2(h, uh, agg, sc4, sh4):
    grid = NP_ // NBLK
    blk = pl.BlockSpec((NBLK, LN), lambda i: (i, 0))
    sspec = pl.BlockSpec((1, LN), lambda i: (0, 0))
    return pl.pallas_call(
        _nb2_body,
        grid=(grid,),
        in_specs=[blk, blk, blk, sspec, sspec],
        out_specs=blk,
        out_shape=jax.ShapeDtypeStruct((NP_, LN), jnp.float32),
        compiler_params=_seq,
    )(h, uh, agg, sc4, sh4)


def _readout_body(wr, w1, b1, w2, b2, w3, b3r, outr):
    t = _silu(jnp.dot(wr[...], w1[...], preferred_element_type=jnp.float32)
              + b1[...])
    t = _silu(jnp.dot(t, w2[...], preferred_element_type=jnp.float32)
              + b2[...])
    z = jnp.dot(t, w3[...], preferred_element_type=jnp.float32) + b3r[0]
    outr[...] = jax.nn.sigmoid(z)


def _readout(w, w18, b18, w28, b28, w38, b3):
    grid = EP_ // EBLK
    blk = pl.BlockSpec((EBLK, LE), lambda i: (i, 0))
    return pl.pallas_call(
        _readout_body,
        grid=(grid,),
        in_specs=[blk,
                  pl.BlockSpec((LE, LE), lambda i: (0, 0)),
                  pl.BlockSpec((1, LE), lambda i: (0, 0)),
                  pl.BlockSpec((LE, LE), lambda i: (0, 0)),
                  pl.BlockSpec((1, LE), lambda i: (0, 0)),
                  pl.BlockSpec((LE, PE), lambda i: (0, 0)),
                  pl.BlockSpec(memory_space=pltpu.SMEM)],
        out_specs=pl.BlockSpec((EBLK, PE), lambda i: (i, 0)),
        out_shape=jax.ShapeDtypeStruct((EP_, PE), jnp.float32),
        compiler_params=_seq,
    )(w, w18, b18, w28, b28, w38, b3)


def _bn_coeffs(sums, count, g, be, p):
    s = sums.reshape(2, p, U).sum(axis=1)
    m = s[0] / count
    v = s[1] / count - m * m
    scale = g * lax.rsqrt(v + 1e-5)
    shift = be - m * scale
    return _tilep(scale, p), _tilep(shift, p)


def _sc_gather(bx, cx, nv, rowi, coli):
    out = jax.ShapeDtypeStruct((E_PAD, U), jnp.float32)

    @functools.partial(
        pl.kernel,
        out_type=[out, out, out],
        mesh=plsc.VectorSubcoreMesh(core_axis_name="c", subcore_axis_name="s"),
        scratch_types=[
            pltpu.VMEM((NSUB, GSUB), jnp.int32),
            pltpu.VMEM((NSUB, GSUB), jnp.int32),
            pltpu.VMEM((GCHUNK, U), jnp.float32),
            pltpu.VMEM((GCHUNK, U), jnp.float32),
            pltpu.VMEM((GCHUNK, U), jnp.float32),
            pltpu.SemaphoreType.DMA,
        ],
        compiler_params=pltpu.CompilerParams(use_tc_tiling_on_sc=False),
    )
    def body(bxr, cxr, nvr, rowr, colr, gbo, gco, gvo,
             rowv, colv, b1, b2, b3, sem):
        c = lax.axis_index("c")
        s = lax.axis_index("s")
        wid = c * SC_SUBCORES + s

        def chunk(ci, carry):
            ibase = wid * (TILE_E // GSUB) + ci * NSUB
            pltpu.sync_copy(rowr.at[pl.ds(ibase, NSUB)], rowv)
            pltpu.sync_copy(colr.at[pl.ds(ibase, NSUB)], colv)
            descs = []
            for j in range(NSUB):
                dst = pl.ds(j * GSUB, GSUB)
                descs.append(pltpu.async_copy(
                    bxr.at[rowv.at[j]], b1.at[dst], sem))
                descs.append(pltpu.async_copy(
                    cxr.at[colv.at[j]], b2.at[dst], sem))
                descs.append(pltpu.async_copy(
                    nvr.at[colv.at[j]], b3.at[dst], sem))
            for d in descs:
                d.wait()
            e0 = wid * TILE_E + ci * GCHUNK
            pltpu.sync_copy(b1, gbo.at[pl.ds(e0, GCHUNK)])
            pltpu.sync_copy(b2, gco.at[pl.ds(e0, GCHUNK)])
            pltpu.sync_copy(b3, gvo.at[pl.ds(e0, GCHUNK)])
            return carry

        lax.fori_loop(0, CHUNKS_PER_TILE, chunk, 0)

    return body(bx, cx, nv, rowi, coli)


def _sc_scatter(msg, rowi, zeros16):
    @functools.partial(
        pl.kernel,
        out_type=jax.ShapeDtypeStruct((N_SC, U), jnp.float32),
        mesh=plsc.VectorSubcoreMesh(core_axis_name="c", subcore_axis_name="s"),
        scratch_types=[
            pltpu.VMEM((NSUB, GSUB), jnp.int32),
            pltpu.VMEM((GCHUNK, U // 2), jnp.float32),
            pltpu.VMEM_SHARED((N_SC, U // 2), jnp.float32),
            pltpu.SemaphoreType.DMA,
        ],
        compiler_params=pltpu.CompilerParams(use_tc_tiling_on_sc=False),
    )
    def body(msgr, rowr, zr, aggo, idxv, mbuf, aggsh, ssem):
        c = lax.axis_index("c")
        s = lax.axis_index("s")
        half = U // 2
        r0 = s * NROW_SC
        pltpu.sync_copy(zr.at[pl.ds(r0, NROW_SC)],
                        aggsh.at[pl.ds(r0, NROW_SC)])
        plsc.subcore_barrier()

        def chunk(ci, carry):
            e0 = s * SCAT_TILE_E + ci * GCHUNK
            pltpu.sync_copy(rowr.at[pl.ds(e0 // GSUB, NSUB)], idxv)
            pltpu.sync_copy(
                msgr.at[pl.ds(e0, GCHUNK), pl.ds(c * half, half)], mbuf)
            descs = []
            for j in range(NSUB):
                descs.append(pltpu.async_copy(
                    mbuf.at[pl.ds(j * GSUB, GSUB)],
                    aggsh.at[idxv.at[j]], ssem, add=True))
            for d in descs:
                d.wait()
            return carry

        lax.fori_loop(0, SCAT_CHUNKS, chunk, 0)
        plsc.subcore_barrier()
        pltpu.sync_copy(
            aggsh.at[pl.ds(r0, NROW_SC)],
            aggo.at[pl.ds(r0, NROW_SC), pl.ds(c * half, half)])

    return body(msg, rowi, zeros16)


def kernel(x, edge_attr, edge_index, W0v, b0v, W0e, b0e, Wu, bu, Wv, bv,
           WA, bA, WB, bB, WC, bC, g_node, be_node, g_edge, be_edge,
           Wp1, bp1, Wp2, bp2, Wp3, bp3):
    row = edge_index[0]
    col = edge_index[1]

    gpad = jnp.zeros((E_PAD - E,), jnp.int32)
    rowg = jnp.concatenate([row, gpad]).reshape(E_PAD // GSUB, GSUB)
    colg = jnp.concatenate([col, gpad]).reshape(E_PAD // GSUB, GSUB)
    junk = N + jnp.arange(E_PAD - E, dtype=jnp.int32) % (N_SC - N)
    rows_sc = jnp.concatenate([row, junk]).reshape(E_PAD // GSUB, GSUB)
    zeros16 = jnp.zeros((N_SC, U // 2), jnp.float32)

    x2 = x.reshape(NP_, PN)
    ea2 = edge_attr.reshape(EP_, PE)

    h = _feat_init(x2, W0v[0], b0v, NP_, NBLK, PN)
    w = _feat_init(ea2, W0e[0], b0e, EP_, EBLK, PE)

    wu4 = jax.vmap(lambda m: _diagp(m, PN))(Wu)
    wv4 = jax.vmap(lambda m: _diagp(m, PN))(Wv)
    wb4 = jax.vmap(lambda m: _diagp(m, PN))(WB)
    wc4 = jax.vmap(lambda m: _diagp(m, PN))(WC)
    wa8 = jax.vmap(lambda m: _diagp(m, PE))(WA)

    for i in range(DEPTH):
        bx, cx, nv, uh = _node_pre(
            h, wb4[i], wc4[i], wv4[i], wu4[i],
            _tilep(bB[i], PN), _tilep(bC[i], PN),
            _tilep(bv[i], PN), _tilep(bu[i], PN))

        gb, gc, gv = _sc_gather(bx.reshape(N, U), cx.reshape(N, U),
                                nv.reshape(N, U), rowg, colg)
        gb = gb.reshape(EP_PAD, LE)
        gc = gc.reshape(EP_PAD, LE)
        gv = gv.reshape(EP_PAD, LE)

        ein, esums = _edge1(w, gb, gc, wa8[i], _tilep(bA[i], PE))
        esc, esh = _bn_coeffs(esums, float(E), g_edge[i], be_edge[i], PE)
        w, msg = _edge2(w, ein, gv, esc, esh)

        agg = _sc_scatter(msg.reshape(E_PAD, U), rows_sc, zeros16)
        agg = agg.reshape(NP_SC, LN)

        nsums = _node_b1(uh, agg)
        nsc, nsh = _bn_coeffs(nsums, float(N), g_node[i], be_node[i], PN)
        h = _node_b2(h, uh, agg, nsc, nsh)

    p = _readout(w, _diagp(Wp1, PE), _tilep(bp1, PE),
                 _diagp(Wp2, PE), _tilep(bp2, PE), _diagp(Wp3, PE), bp3)
    return p.reshape(E)


# speedup vs baseline: 1.4120x; 1.0192x over previous
import functools
import math

import jax
import jax.numpy as jnp
from jax import lax
from jax.experimental import pallas as pl
from jax.experimental.pallas import tpu as pltpu
from jax.experimental.pallas import tpu_sc as plsc

N = 100000
E = 1600000
U = 32
DEPTH = 12

PE = 4
LE = PE * U
PN = 4
LN = PN * U
NP_ = N // PN
EP_ = E // PE

NBLK = 1000
EBLK = 2000

SC_CORES = 2
SC_SUBCORES = 16
SC_WORKERS = SC_CORES * SC_SUBCORES
GCHUNK = 1024
GSUB = 128
NSUB = GCHUNK // GSUB
CHUNKS_PER_TILE = 49
TILE_E = CHUNKS_PER_TILE * GCHUNK
E_PAD = SC_WORKERS * TILE_E
EP_PAD = E_PAD // PE
SCAT_TILE_E = E_PAD // SC_SUBCORES
SCAT_CHUNKS = SCAT_TILE_E // GCHUNK
JUNK = N
N_SC = 100352
NP_SC = N_SC // PN
NROW_SC = N_SC // SC_SUBCORES

_seq = pltpu.CompilerParams(dimension_semantics=("arbitrary",))


def _diagp(w, p):
    return jnp.kron(jnp.eye(p, dtype=w.dtype), w)


def _tilep(b, p):
    return jnp.tile(b, p).reshape(1, p * U)


def _silu(z):
    return z * jax.nn.sigmoid(z)


def _init_body(lanes, xr, wr, br, outr):
    xb = xr[...]
    xe = jnp.broadcast_to(xb[:, :, None], xb.shape + (U,)).reshape(
        xb.shape[0], lanes)
    outr[...] = _silu(xe * wr[...] + br[...])


def _feat_init(x2d, w_row, bias, nrows, blk, p):
    grid = nrows // blk
    lanes = p * U
    return pl.pallas_call(
        functools.partial(_init_body, lanes),
        grid=(grid,),
        in_specs=[
            pl.BlockSpec((blk, p), lambda i: (i, 0)),
            pl.BlockSpec((1, lanes), lambda i: (0, 0)),
            pl.BlockSpec((1, lanes), lambda i: (0, 0)),
        ],
        out_specs=pl.BlockSpec((blk, lanes), lambda i: (i, 0)),
        out_shape=jax.ShapeDtypeStruct((nrows, lanes), jnp.float32),
        compiler_params=_seq,
    )(x2d, _tilep(w_row, p), _tilep(bias, p))


def _node_pre_body(hr, wb, wc, wv, wu, bb, bc, bv, bu, bxr, cxr, nvr, uhr):
    h = hr[...]
    bxr[...] = jnp.dot(h, wb[...], preferred_element_type=jnp.float32) + bb[...]
    cxr[...] = jnp.dot(h, wc[...], preferred_element_type=jnp.float32) + bc[...]
    nvr[...] = jnp.dot(h, wv[...], preferred_element_type=jnp.float32) + bv[...]
    uhr[...] = jnp.dot(h, wu[...], preferred_element_type=jnp.float32) + bu[...]


def _node_pre(h, wb4, wc4, wv4, wu4, bb4, bc4, bv4, bu4):
    grid = NP_ // NBLK
    blk = pl.BlockSpec((NBLK, LN), lambda i: (i, 0))
    wspec = pl.BlockSpec((LN, LN), lambda i: (0, 0))
    bspec = pl.BlockSpec((1, LN), lambda i: (0, 0))
    out = jax.ShapeDtypeStruct((NP_, LN), jnp.float32)
    return pl.pallas_call(
        _node_pre_body,
        grid=(grid,),
        in_specs=[blk, wspec, wspec, wspec, wspec, bspec, bspec, bspec, bspec],
        out_specs=[blk, blk, blk, blk],
        out_shape=[out, out, out, out],
        compiler_params=_seq,
    )(h, wb4, wc4, wv4, wu4, bb4, bc4, bv4, bu4)


def _e1_body(wr, gbr, gcr, wa, ba, einr, sumr):
    i = pl.program_id(0)
    ein = (jnp.dot(wr[...], wa[...], preferred_element_type=jnp.float32)
           + ba[...] + gbr[...] + gcr[...])
    einr[...] = ein

    @pl.when(i == 0)
    def _():
        sumr[...] = jnp.zeros_like(sumr)

    s1 = jnp.sum(ein, axis=0, keepdims=True)
    s2 = jnp.sum(ein * ein, axis=0, keepdims=True)
    sumr[...] += jnp.concatenate([s1, s2], axis=0)


def _edge1(w, gb, gc, wa8, ba8):
    grid = EP_ // EBLK
    blk = pl.BlockSpec((EBLK, LE), lambda i: (i, 0))
    return pl.pallas_call(
        _e1_body,
        grid=(grid,),
        in_specs=[blk, blk, blk,
                  pl.BlockSpec((LE, LE), lambda i: (0, 0)),
                  pl.BlockSpec((1, LE), lambda i: (0, 0))],
        out_specs=[blk, pl.BlockSpec((2, LE), lambda i: (0, 0))],
        out_shape=[jax.ShapeDtypeStruct((EP_, LE), jnp.float32),
                   jax.ShapeDtypeStruct((2, LE), jnp.float32)],
        compiler_params=_seq,
    )(w, gb, gc, wa8, ba8)


def _e2_body(wr, einr, gvr, scr, shr, wnr, msgr):
    tmp = jnp.maximum(einr[...] * scr[...] + shr[...], 0.0)
    wn = wr[...] + tmp
    wnr[...] = wn
    msgr[...] = jax.nn.sigmoid(wn) * gvr[...]


def _edge2(w, ein, gv, sc8, sh8):
    grid = EP_ // EBLK
    blk = pl.BlockSpec((EBLK, LE), lambda i: (i, 0))
    sspec = pl.BlockSpec((1, LE), lambda i: (0, 0))
    return pl.pallas_call(
        _e2_body,
        grid=(grid,),
        in_specs=[blk, blk, blk, sspec, sspec],
        out_specs=[blk, blk],
        out_shape=[jax.ShapeDtypeStruct((EP_, LE), jnp.float32),
                   jax.ShapeDtypeStruct((EP_PAD, LE), jnp.float32)],
        compiler_params=_seq,
    )(w, ein, gv, sc8, sh8)


def _nb1_body(uhr, aggr, sumr):
    i = pl.program_id(0)
    t = uhr[...] + aggr[...]

    @pl.when(i == 0)
    def _():
        sumr[...] = jnp.zeros_like(sumr)

    s1 = jnp.sum(t, axis=0, keepdims=True)
    s2 = jnp.sum(t * t, axis=0, keepdims=True)
    sumr[...] += jnp.concatenate([s1, s2], axis=0)


def _node_b1(uh, agg):
    grid = NP_ // NBLK
    blk = pl.BlockSpec((NBLK, LN), lambda i: (i, 0))
    return pl.pallas_call(
        _nb1_body,
        grid=(grid,),
        in_specs=[blk, blk],
        out_specs=pl.BlockSpec((2, LN), lambda i: (0, 0)),
        out_shape=jax.ShapeDtypeStruct((2, LN), jnp.float32),
        compiler_params=_seq,
    )(uh, agg)


def _nb2_body(hr, uhr, aggr, scr, shr, outr):
    t = uhr[...] + aggr[...]
    outr[...] = hr[...] + jnp.maximum(t * scr[...] + shr[...], 0.0)


def _node_b2(h, uh, agg, sc4, sh4):
    grid = NP_ // NBLK
    blk = pl.BlockSpec((NBLK, LN), lambda i: (i, 0))
    sspec = pl.BlockSpec((1, LN), lambda i: (0, 0))
    return pl.pallas_call(
        _nb2_body,
        grid=(grid,),
        in_specs=[blk, blk, blk, sspec, sspec],
        out_specs=blk,
        out_shape=jax.ShapeDtypeStruct((NP_, LN), jnp.float32),
        compiler_params=_seq,
    )(h, uh, agg, sc4, sh4)


def _readout_body(wr, w1, b1, w2, b2, w3, b3r, outr):
    t = _silu(jnp.dot(wr[...], w1[...], preferred_element_type=jnp.float32)
              + b1[...])
    t = _silu(jnp.dot(t, w2[...], preferred_element_type=jnp.float32)
              + b2[...])
    z = jnp.dot(t, w3[...], preferred_element_type=jnp.float32) + b3r[0]
    outr[...] = jax.nn.sigmoid(z)


def _readout(w, w18, b18, w28, b28, w38, b3):
    grid = EP_ // EBLK
    blk = pl.BlockSpec((EBLK, LE), lambda i: (i, 0))
    return pl.pallas_call(
        _readout_body,
        grid=(grid,),
        in_specs=[blk,
                  pl.BlockSpec((LE, LE), lambda i: (0, 0)),
                  pl.BlockSpec((1, LE), lambda i: (0, 0)),
                  pl.BlockSpec((LE, LE), lambda i: (0, 0)),
                  pl.BlockSpec((1, LE), lambda i: (0, 0)),
                  pl.BlockSpec((LE, PE), lambda i: (0, 0)),
                  pl.BlockSpec(memory_space=pltpu.SMEM)],
        out_specs=pl.BlockSpec((EBLK, PE), lambda i: (i, 0)),
        out_shape=jax.ShapeDtypeStruct((EP_, PE), jnp.float32),
        compiler_params=_seq,
    )(w, w18, b18, w28, b28, w38, b3)


def _bn_coeffs(sums, count, g, be, p):
    s = sums.reshape(2, p, U).sum(axis=1)
    m = s[0] / count
    v = s[1] / count - m * m
    scale = g * lax.rsqrt(v + 1e-5)
    shift = be - m * scale
    return _tilep(scale, p), _tilep(shift, p)


def _sc_gather(bx, cx, nv, rowi, coli):
    out = jax.ShapeDtypeStruct((E_PAD, U), jnp.float32)

    @functools.partial(
        pl.kernel,
        out_type=[out, out, out],
        mesh=plsc.VectorSubcoreMesh(core_axis_name="c", subcore_axis_name="s"),
        scratch_types=[
            pltpu.VMEM((NSUB, GSUB), jnp.int32),
            pltpu.VMEM((NSUB, GSUB), jnp.int32),
            pltpu.VMEM((GCHUNK, U), jnp.float32),
            pltpu.VMEM((GCHUNK, U), jnp.float32),
            pltpu.VMEM((GCHUNK, U), jnp.float32),
            pltpu.SemaphoreType.DMA,
            pltpu.SemaphoreType.DMA,
        ],
        compiler_params=pltpu.CompilerParams(use_tc_tiling_on_sc=False),
    )
    def body(bxr, cxr, nvr, rowr, colr, gbo, gco, gvo,
             rowv, colv, b1, b2, b3, sem, wsem):
        c = lax.axis_index("c")
        s = lax.axis_index("s")
        wid = c * SC_SUBCORES + s

        def chunk(ci, carry):
            ibase = wid * (TILE_E // GSUB) + ci * NSUB
            pltpu.sync_copy(rowr.at[pl.ds(ibase, NSUB)], rowv)
            pltpu.sync_copy(colr.at[pl.ds(ibase, NSUB)], colv)

            @pl.when(ci > 0)
            def _():
                for buf, dsthbm in ((b1, gbo), (b2, gco), (b3, gvo)):
                    pltpu.make_async_copy(
                        dsthbm.at[pl.ds(0, GCHUNK)], buf, wsem).wait()

            descs = []
            for j in range(NSUB):
                dst = pl.ds(j * GSUB, GSUB)
                descs.append(pltpu.async_copy(
                    bxr.at[rowv.at[j]], b1.at[dst], sem))
                descs.append(pltpu.async_copy(
                    cxr.at[colv.at[j]], b2.at[dst], sem))
                descs.append(pltpu.async_copy(
                    nvr.at[colv.at[j]], b3.at[dst], sem))
            for d in descs:
                d.wait()
            e0 = wid * TILE_E + ci * GCHUNK
            pltpu.async_copy(b1, gbo.at[pl.ds(e0, GCHUNK)], wsem)
            pltpu.async_copy(b2, gco.at[pl.ds(e0, GCHUNK)], wsem)
            pltpu.async_copy(b3, gvo.at[pl.ds(e0, GCHUNK)], wsem)
            return carry

        lax.fori_loop(0, CHUNKS_PER_TILE, chunk, 0)
        for buf, dsthbm in ((b1, gbo), (b2, gco), (b3, gvo)):
            pltpu.make_async_copy(
                dsthbm.at[pl.ds(0, GCHUNK)], buf, wsem).wait()

    return body(bx, cx, nv, rowi, coli)


def _sc_scatter(msg, rowi, zeros16):
    @functools.partial(
        pl.kernel,
        out_type=jax.ShapeDtypeStruct((N_SC, U), jnp.float32),
        mesh=plsc.VectorSubcoreMesh(core_axis_name="c", subcore_axis_name="s"),
        scratch_types=[
            pltpu.VMEM((NSUB, GSUB), jnp.int32),
            pltpu.VMEM((GCHUNK, U // 2), jnp.float32),
            pltpu.VMEM_SHARED((N_SC, U // 2), jnp.float32),
            pltpu.SemaphoreType.DMA,
        ],
        compiler_params=pltpu.CompilerParams(use_tc_tiling_on_sc=False),
    )
    def body(msgr, rowr, zr, aggo, idxv, mbuf, aggsh, ssem):
        c = lax.axis_index("c")
        s = lax.axis_index("s")
        half = U // 2
        r0 = s * NROW_SC
        pltpu.sync_copy(zr.at[pl.ds(r0, NROW_SC)],
                        aggsh.at[pl.ds(r0, NROW_SC)])
        plsc.subcore_barrier()

        def chunk(ci, carry):
            e0 = s * SCAT_TILE_E + ci * GCHUNK
            pltpu.sync_copy(rowr.at[pl.ds(e0 // GSUB, NSUB)], idxv)
            pltpu.sync_copy(
                msgr.at[pl.ds(e0, GCHUNK), pl.ds(c * half, half)], mbuf)
            descs = []
            for j in range(NSUB):
                descs.append(pltpu.async_copy(
                    mbuf.at[pl.ds(j * GSUB, GSUB)],
                    aggsh.at[idxv.at[j]], ssem, add=True))
            for d in descs:
                d.wait()
            return carry

        lax.fori_loop(0, SCAT_CHUNKS, chunk, 0)
        plsc.subcore_barrier()
        pltpu.sync_copy(
            aggsh.at[pl.ds(r0, NROW_SC)],
            aggo.at[pl.ds(r0, NROW_SC), pl.ds(c * half, half)])

    return body(msg, rowi, zeros16)


def kernel(x, edge_attr, edge_index, W0v, b0v, W0e, b0e, Wu, bu, Wv, bv,
           WA, bA, WB, bB, WC, bC, g_node, be_node, g_edge, be_edge,
           Wp1, bp1, Wp2, bp2, Wp3, bp3):
    row = edge_index[0]
    col = edge_index[1]

    gpad = jnp.zeros((E_PAD - E,), jnp.int32)
    rowg = jnp.concatenate([row, gpad]).reshape(E_PAD // GSUB, GSUB)
    colg = jnp.concatenate([col, gpad]).reshape(E_PAD // GSUB, GSUB)
    junk = N + jnp.arange(E_PAD - E, dtype=jnp.int32) % (N_SC - N)
    rows_sc = jnp.concatenate([row, junk]).reshape(E_PAD // GSUB, GSUB)
    zeros16 = jnp.zeros((N_SC, U // 2), jnp.float32)

    x2 = x.reshape(NP_, PN)
    ea2 = edge_attr.reshape(EP_, PE)

    h = _feat_init(x2, W0v[0], b0v, NP_, NBLK, PN)
    w = _feat_init(ea2, W0e[0], b0e, EP_, EBLK, PE)

    wu4 = jax.vmap(lambda m: _diagp(m, PN))(Wu)
    wv4 = jax.vmap(lambda m: _diagp(m, PN))(Wv)
    wb4 = jax.vmap(lambda m: _diagp(m, PN))(WB)
    wc4 = jax.vmap(lambda m: _diagp(m, PN))(WC)
    wa8 = jax.vmap(lambda m: _diagp(m, PE))(WA)

    for i in range(DEPTH):
        bx, cx, nv, uh = _node_pre(
            h, wb4[i], wc4[i], wv4[i], wu4[i],
            _tilep(bB[i], PN), _tilep(bC[i], PN),
            _tilep(bv[i], PN), _tilep(bu[i], PN))

        gb, gc, gv = _sc_gather(bx.reshape(N, U), cx.reshape(N, U),
                                nv.reshape(N, U), rowg, colg)
        gb = gb.reshape(EP_PAD, LE)
        gc = gc.reshape(EP_PAD, LE)
        gv = gv.reshape(EP_PAD, LE)

        ein, esums = _edge1(w, gb, gc, wa8[i], _tilep(bA[i], PE))
        esc, esh = _bn_coeffs(esums, float(E), g_edge[i], be_edge[i], PE)
        w, msg = _edge2(w, ein, gv, esc, esh)

        agg = _sc_scatter(msg.reshape(E_PAD, U), rows_sc, zeros16)
        agg = agg.reshape(NP_SC, LN)

        nsums = _node_b1(uh, agg)
        nsc, nsh = _bn_coeffs(nsums, float(N), g_node[i], be_node[i], PN)
        h = _node_b2(h, uh, agg, nsc, nsh)

    p = _readout(w, _diagp(Wp1, PE), _tilep(bp1, PE),
                 _diagp(Wp2, PE), _tilep(bp2, PE), _diagp(Wp3, PE), bp3)
    return p.reshape(E)
